# Initial kernel scaffold; baseline (speedup 1.0000x reference)
import jax
import jax.numpy as jnp
from jax.experimental import pallas as pl


def kernel(x, Wr, br, We, be):
    raise NotImplementedError("write your pallas kernel here")


# speedup vs baseline: 5.5240x; 5.5240x over previous
import jax
import jax.numpy as jnp
from jax.experimental import pallas as pl

HIDDEN = 2048
NUM_EXPERTS = 8
ROWS_BLK = 1024


def _stats_kernel(x_ref, wr_ref, br_ref, g_ref, s_ref):
    i = pl.program_id(0)
    xb = x_ref[...]
    logits = jax.lax.dot_general(
        xb, wr_ref[...], (((1,), (0,)), ((), ())),
        preferred_element_type=jnp.float32,
        precision=jax.lax.Precision.HIGHEST,
    ) + br_ref[...]
    m1 = jnp.max(logits, axis=-1, keepdims=True)
    am = jnp.argmax(logits, axis=-1)[:, None]
    col = jax.lax.broadcasted_iota(jnp.int32, logits.shape, 1)
    l2 = jnp.max(jnp.where(col == am, -jnp.inf, logits), axis=-1, keepdims=True)
    denom = jnp.sum(jnp.exp(logits - m1), axis=-1, keepdims=True)
    g_ref[...] = (1.0 + jnp.exp(l2 - m1)) / denom

    part = jnp.sum(xb, axis=0, keepdims=True)

    @pl.when(i == 0)
    def _():
        s_ref[...] = part

    @pl.when(i != 0)
    def _():
        s_ref[...] += part


def _v_kernel(s_ref, we_ref, be_ref, v_ref):
    v_ref[...] = jax.lax.dot_general(
        s_ref[...], we_ref[...], (((1,), (0,)), ((), ())),
        preferred_element_type=jnp.float32,
        precision=jax.lax.Precision.HIGHEST,
    ) + float(NUM_EXPERTS) * be_ref[...]


def _outer_kernel(g_ref, v_ref, y_ref):
    y_ref[...] = g_ref[...] * v_ref[...]


def kernel(x, Wr, br, We, be):
    b, seq, h = x.shape
    n = b * seq
    xf = x.reshape(n, h)
    nblk = n // ROWS_BLK

    g, s = pl.pallas_call(
        _stats_kernel,
        grid=(nblk,),
        in_specs=[
            pl.BlockSpec((ROWS_BLK, h), lambda i: (i, 0)),
            pl.BlockSpec((h, NUM_EXPERTS), lambda i: (0, 0)),
            pl.BlockSpec((1, NUM_EXPERTS), lambda i: (0, 0)),
        ],
        out_specs=[
            pl.BlockSpec((ROWS_BLK, 1), lambda i: (i, 0)),
            pl.BlockSpec((1, h), lambda i: (0, 0)),
        ],
        out_shape=[
            jax.ShapeDtypeStruct((n, 1), jnp.float32),
            jax.ShapeDtypeStruct((1, h), jnp.float32),
        ],
    )(xf, Wr, br.reshape(1, NUM_EXPERTS))

    v = pl.pallas_call(
        _v_kernel,
        in_specs=[
            pl.BlockSpec((1, h), lambda: (0, 0)),
            pl.BlockSpec((h, h), lambda: (0, 0)),
            pl.BlockSpec((1, h), lambda: (0, 0)),
        ],
        out_specs=pl.BlockSpec((1, h), lambda: (0, 0)),
        out_shape=jax.ShapeDtypeStruct((1, h), jnp.float32),
    )(s, We, be.reshape(1, h))

    y = pl.pallas_call(
        _outer_kernel,
        grid=(nblk,),
        in_specs=[
            pl.BlockSpec((ROWS_BLK, 1), lambda i: (i, 0)),
            pl.BlockSpec((1, h), lambda i: (0, 0)),
        ],
        out_specs=pl.BlockSpec((ROWS_BLK, h), lambda i: (i, 0)),
        out_shape=jax.ShapeDtypeStruct((n, h), jnp.float32),
    )(g, v)

    return y.reshape(b, seq, h)


# speedup vs baseline: 10.7916x; 1.9536x over previous
import jax
import jax.numpy as jnp
from jax.experimental import pallas as pl
from jax.experimental.pallas import tpu as pltpu

HIDDEN = 2048
NUM_EXPERTS = 8
ROWS_BLK = 1024


def _fused_kernel(x_ref, wr_ref, br_ref, we_ref, be_ref, y_ref,
                  g_scr, s_scr, v_scr):
    p = pl.program_id(0)
    i = pl.program_id(1)

    @pl.when(p == 0)
    def _():
        xb = x_ref[...]
        logits = jax.lax.dot_general(
            xb, wr_ref[...], (((1,), (0,)), ((), ())),
            preferred_element_type=jnp.float32,
        ) + br_ref[...]
        m1 = jnp.max(logits, axis=-1, keepdims=True)
        am = jnp.argmax(logits, axis=-1)[:, None]
        col = jax.lax.broadcasted_iota(jnp.int32, logits.shape, 1)
        l2 = jnp.max(jnp.where(col == am, -jnp.inf, logits), axis=-1,
                     keepdims=True)
        denom = jnp.sum(jnp.exp(logits - m1), axis=-1, keepdims=True)
        g_scr[pl.ds(i * ROWS_BLK, ROWS_BLK), :] = \
            (1.0 + jnp.exp(l2 - m1)) / denom

        part = jnp.sum(xb, axis=0, keepdims=True)

        @pl.when(i == 0)
        def _():
            s_scr[...] = part

        @pl.when(i != 0)
        def _():
            s_scr[...] += part

    @pl.when((p == 1) & (i == 0))
    def _():
        s_col = jnp.transpose(s_scr[...])
        v_scr[...] = jnp.sum(we_ref[...] * s_col, axis=0, keepdims=True) \
            + float(NUM_EXPERTS) * be_ref[...]

    @pl.when(p == 1)
    def _():
        y_ref[...] = g_scr[pl.ds(i * ROWS_BLK, ROWS_BLK), :] * v_scr[...]


def kernel(x, Wr, br, We, be):
    b, seq, h = x.shape
    n = b * seq
    xf = x.reshape(n, h)
    nblk = n // ROWS_BLK

    y = pl.pallas_call(
        _fused_kernel,
        grid=(2, nblk),
        in_specs=[
            pl.BlockSpec((ROWS_BLK, h),
                         lambda p, i: (jnp.where(p == 0, i, nblk - 1), 0)),
            pl.BlockSpec((h, NUM_EXPERTS), lambda p, i: (0, 0)),
            pl.BlockSpec((1, NUM_EXPERTS), lambda p, i: (0, 0)),
            pl.BlockSpec((h, h), lambda p, i: (0, 0)),
            pl.BlockSpec((1, h), lambda p, i: (0, 0)),
        ],
        out_specs=pl.BlockSpec((ROWS_BLK, h),
                               lambda p, i: (jnp.where(p == 0, 0, i), 0)),
        out_shape=jax.ShapeDtypeStruct((n, h), jnp.float32),
        scratch_shapes=[
            pltpu.VMEM((n, 1), jnp.float32),
            pltpu.VMEM((1, h), jnp.float32),
            pltpu.VMEM((1, h), jnp.float32),
        ],
    )(xf, Wr, br.reshape(1, NUM_EXPERTS), We, be.reshape(1, h))

    return y.reshape(b, seq, h)
